# baseline (speedup 1.0000x reference)
import functools

import jax
import jax.numpy as jnp
from jax import lax
from jax.experimental import pallas as pl
from jax.experimental.pallas import tpu as pltpu
from jax.experimental.pallas import tpu_sc as plsc

_NUM_STYLES = 64
_DIM = 192
_BATCH = 16384
_LANES = 16


def _table_body(emb_ref, w_ref, b_ref, p_ref):
    p_ref[...] = (
        lax.dot_general(
            emb_ref[...],
            w_ref[...],
            (((1,), (1,)), ((), ())),
            preferred_element_type=jnp.float32,
        )
        + b_ref[...]
    )


def _make_sc_kernel():
    info = plsc.get_sparse_core_info()
    nc, ns = info.num_cores, info.num_subcores
    nw = nc * ns
    bpw = _BATCH // nw
    nch = 4
    ch = bpw // nch
    chw = ch * _DIM
    nvec = _DIM // _LANES

    mesh = plsc.VectorSubcoreMesh(core_axis_name="c", subcore_axis_name="s")

    @functools.partial(
        pl.kernel,
        mesh=mesh,
        compiler_params=pltpu.CompilerParams(
            needs_layout_passes=False, use_tc_tiling_on_sc=False
        ),
        out_type=jax.ShapeDtypeStruct((_BATCH * _DIM,), jnp.float32),
        scratch_types=[
            pltpu.VMEM((nch, ch), jnp.int32),
            pltpu.VMEM((bpw,), jnp.float32),
            pltpu.VMEM((2, ch, _DIM), jnp.float32),
            pltpu.VMEM((2, chw), jnp.float32),
            pltpu.SemaphoreType.DMA,
            pltpu.SemaphoreType.DMA,
            pltpu.SemaphoreType.DMA,
            pltpu.SemaphoreType.DMA,
        ],
    )
    def sc_kernel(
        sid_hbm, exa_hbm, p_hbm, out_hbm, idx_v, exa_v, gbuf, sbuf, g0, g1, o0, o1
    ):
        wid = lax.axis_index("s") * nc + lax.axis_index("c")
        base = wid * bpw
        gsems = (g0, g1)
        osems = (o0, o1)
        for k in range(nch):
            pltpu.sync_copy(sid_hbm.at[pl.ds(base + k * ch, ch)], idx_v.at[k])
        gathers = [
            pltpu.async_copy(p_hbm.at[idx_v.at[0]], gbuf.at[0], gsems[0]),
        ]
        pltpu.sync_copy(exa_hbm.at[pl.ds(base, bpw)], exa_v)

        stores = [None, None]
        for k in range(nch):
            s = k % 2
            if k + 1 < nch:
                gathers.append(
                    pltpu.async_copy(
                        p_hbm.at[idx_v.at[k + 1]], gbuf.at[(k + 1) % 2], gsems[(k + 1) % 2]
                    )
                )
            gathers[k].wait()
            if stores[s] is not None:
                stores[s].wait()

            def body(r, _):
                e = plsc.load_gather(
                    exa_v, [jnp.full((_LANES,), k * ch + r, jnp.int32)]
                )
                for j in range(nvec):
                    src = gbuf[s, r, pl.ds(j * _LANES, _LANES)]
                    sbuf[s, pl.ds(r * _DIM + j * _LANES, _LANES)] = src * e
                return _

            lax.fori_loop(0, ch, body, 0, unroll=2)
            stores[s] = pltpu.async_copy(
                sbuf.at[s], out_hbm.at[pl.ds(base * _DIM + k * chw, chw)], osems[s]
            )
        for st in stores:
            st.wait()

    return sc_kernel


_SC_KERNEL = _make_sc_kernel()


def kernel(style_id, exaggeration, emb, W, b):
    p = pl.pallas_call(
        _table_body,
        out_shape=jax.ShapeDtypeStruct((_NUM_STYLES, _DIM), jnp.float32),
    )(emb, W, b.reshape(1, _DIM))
    out = _SC_KERNEL(style_id, exaggeration, p)
    return out.reshape(_BATCH, _DIM)
